# baseline (speedup 1.0000x reference)
import functools

import jax
import jax.numpy as jnp
from jax import lax
from jax.experimental import pallas as pl
from jax.experimental.pallas import tpu as pltpu
from jax.experimental.pallas import tpu_sc as plsc

EMB_DIM = 64
PAD_DIM = 128
SUB = 80
CHUNK = 320
NBUF = 2


@functools.lru_cache(maxsize=None)
def _build(B, NC, NS):
  NW = NC * NS
  b_per_w = B // NW
  n_sub = b_per_w // SUB
  n_chunks = b_per_w // CHUNK
  sub_per_chunk = CHUNK // SUB
  n_groups = n_chunks // NBUF
  mesh = plsc.VectorSubcoreMesh(core_axis_name="c", subcore_axis_name="s")

  n_sent = B // 20
  seq = 20

  @functools.partial(
      pl.kernel, mesh=mesh,
      compiler_params=pltpu.CompilerParams(use_tc_tiling_on_sc=False),
      out_type=jax.ShapeDtypeStruct((n_sent, 24, PAD_DIM), jnp.float32),
      scratch_types=(
          [pltpu.VMEM((n_sub, SUB), jnp.int32)]
          + [pltpu.VMEM((CHUNK, EMB_DIM), jnp.float32) for _ in range(NBUF)]
          + [pltpu.SemaphoreType.DMA for _ in range(2 * NBUF)]
      ),
  )
  def gather_kernel(table_hbm, idx_hbm, out_hbm, idx_v, *rest):
    rows = rest[:NBUF]
    gsem = rest[NBUF:2 * NBUF]
    ssem = rest[2 * NBUF:]
    wid = lax.axis_index("s") * NC + lax.axis_index("c")
    sent_base = wid * (b_per_w // seq)
    sent_per_chunk = CHUNK // seq

    pltpu.sync_copy(idx_hbm.at[wid], idx_v)

    def start_gather(g, b):
      for k in range(sub_per_chunk):
        pltpu.async_copy(table_hbm.at[idx_v.at[g * sub_per_chunk + k]],
                         rows[b].at[pl.ds(k * SUB, SUB)], gsem[b])

    def wait_gather(g, b):
      for k in range(sub_per_chunk):
        pltpu.make_async_copy(table_hbm.at[idx_v.at[g * sub_per_chunk + k]],
                              rows[b].at[pl.ds(k * SUB, SUB)], gsem[b]).wait()

    def start_store(g, b):
      s0 = sent_base + g * sent_per_chunk
      for s in range(sent_per_chunk):
        pltpu.async_copy(
            rows[b].at[pl.ds(s * seq, seq)],
            out_hbm.at[s0 + s, pl.ds(0, seq), pl.ds(0, EMB_DIM)], ssem[b])

    def wait_store(g, b):
      s0 = sent_base + g * sent_per_chunk
      for s in range(sent_per_chunk):
        pltpu.make_async_copy(
            rows[b].at[pl.ds(s * seq, seq)],
            out_hbm.at[s0 + s, pl.ds(0, seq), pl.ds(0, EMB_DIM)],
            ssem[b]).wait()

    for b in range(NBUF):
      start_gather(b, b)

    def body(go, _):
      for b in range(NBUF):
        g = go * NBUF + b
        wait_gather(g, b)
        start_store(g, b)
      for b in range(NBUF):
        g = go * NBUF + b
        wait_store(g, b)
        start_gather(g + NBUF, b)
      return _

    lax.fori_loop(0, n_groups - 1, body, None)

    for b in range(NBUF):
      g = (n_groups - 1) * NBUF + b
      wait_gather(g, b)
      start_store(g, b)
    for b in range(NBUF):
      g = (n_groups - 1) * NBUF + b
      wait_store(g, b)

  return gather_kernel


def kernel(input_, weight):
  info = plsc.get_sparse_core_info()
  NC, NS = info.num_cores, info.num_subcores
  B = input_.size
  idx = input_.reshape(-1).astype(jnp.int32)
  idx3 = idx.reshape(NC * NS, -1, SUB)
  out = _build(B, NC, NS)(weight, idx3)
  return out[:, :input_.shape[1], :EMB_DIM]


# speedup vs baseline: 1.0068x; 1.0068x over previous
import functools

import jax
import jax.numpy as jnp
from jax import lax
from jax.experimental import pallas as pl
from jax.experimental.pallas import tpu as pltpu
from jax.experimental.pallas import tpu_sc as plsc

EMB_DIM = 64
PAD_DIM = 128
SUB = 80
CHUNK = 320
NBUF = 2


@functools.lru_cache(maxsize=None)
def _build(B, NC, NS):
  NW = NC * NS
  b_per_w = B // NW
  n_sub = b_per_w // SUB
  n_chunks = b_per_w // CHUNK
  sub_per_chunk = CHUNK // SUB
  n_groups = n_chunks // NBUF
  mesh = plsc.VectorSubcoreMesh(core_axis_name="c", subcore_axis_name="s")

  n_sent = B // 20
  seq = 20

  @functools.partial(
      pl.kernel, mesh=mesh,
      compiler_params=pltpu.CompilerParams(use_tc_tiling_on_sc=False),
      out_type=jax.ShapeDtypeStruct((n_sent, 24, PAD_DIM), jnp.float32),
      scratch_types=(
          [pltpu.VMEM((n_sub, SUB), jnp.int32)]
          + [pltpu.VMEM((CHUNK, PAD_DIM), jnp.float32) for _ in range(NBUF)]
          + [pltpu.SemaphoreType.DMA for _ in range(2 * NBUF)]
      ),
  )
  def gather_kernel(table_hbm, idx_hbm, out_hbm, idx_v, *rest):
    rows = rest[:NBUF]
    gsem = rest[NBUF:2 * NBUF]
    ssem = rest[2 * NBUF:]
    wid = lax.axis_index("s") * NC + lax.axis_index("c")
    sent_base = wid * (b_per_w // seq)
    sent_per_chunk = CHUNK // seq

    pltpu.sync_copy(idx_hbm.at[wid], idx_v)

    def start_gather(g, b):
      for k in range(sub_per_chunk):
        pltpu.async_copy(table_hbm.at[idx_v.at[g * sub_per_chunk + k]],
                         rows[b].at[pl.ds(k * SUB, SUB)], gsem[b])

    def wait_gather(g, b):
      for k in range(sub_per_chunk):
        pltpu.make_async_copy(table_hbm.at[idx_v.at[g * sub_per_chunk + k]],
                              rows[b].at[pl.ds(k * SUB, SUB)], gsem[b]).wait()

    def start_store(g, b):
      s0 = sent_base + g * sent_per_chunk
      for s in range(sent_per_chunk):
        pltpu.async_copy(
            rows[b].at[pl.ds(s * seq, seq), pl.ds(0, EMB_DIM)],
            out_hbm.at[s0 + s, pl.ds(0, seq), pl.ds(0, EMB_DIM)], ssem[b])

    def wait_store(g, b):
      s0 = sent_base + g * sent_per_chunk
      for s in range(sent_per_chunk):
        pltpu.make_async_copy(
            rows[b].at[pl.ds(s * seq, seq), pl.ds(0, EMB_DIM)],
            out_hbm.at[s0 + s, pl.ds(0, seq), pl.ds(0, EMB_DIM)],
            ssem[b]).wait()

    for b in range(NBUF):
      start_gather(b, b)

    def body(go, _):
      for b in range(NBUF):
        g = go * NBUF + b
        wait_gather(g, b)
        start_store(g, b)
      for b in range(NBUF):
        g = go * NBUF + b
        wait_store(g, b)
        start_gather(g + NBUF, b)
      return _

    lax.fori_loop(0, n_groups - 1, body, None)

    for b in range(NBUF):
      g = (n_groups - 1) * NBUF + b
      wait_gather(g, b)
      start_store(g, b)
    for b in range(NBUF):
      g = (n_groups - 1) * NBUF + b
      wait_store(g, b)

  return gather_kernel


def kernel(input_, weight):
  info = plsc.get_sparse_core_info()
  NC, NS = info.num_cores, info.num_subcores
  B = input_.size
  wpad = jnp.pad(weight, ((0, 0), (0, PAD_DIM - EMB_DIM)))
  idx = input_.reshape(-1).astype(jnp.int32)
  idx3 = idx.reshape(NC * NS, -1, SUB)
  out = _build(B, NC, NS)(wpad, idx3)
  return out[:, :input_.shape[1], :EMB_DIM]


# speedup vs baseline: 1.0069x; 1.0002x over previous
import functools

import jax
import jax.numpy as jnp
from jax import lax
from jax.experimental import pallas as pl
from jax.experimental.pallas import tpu as pltpu
from jax.experimental.pallas import tpu_sc as plsc

EMB_DIM = 64
PAD_DIM = 128
SUB = 80
CHUNK = 320
NBUF = 2


@functools.lru_cache(maxsize=None)
def _build(B, seq, NC, NS):
  NW = NC * NS
  b_per_w = B // NW
  n_sub = b_per_w // SUB
  n_chunks = b_per_w // CHUNK
  sub_per_chunk = CHUNK // SUB
  n_groups = n_chunks // NBUF
  mesh = plsc.VectorSubcoreMesh(core_axis_name="c", subcore_axis_name="s")

  n_sent = B // seq
  seq_pad = (seq + 7) // 8 * 8

  @functools.partial(
      pl.kernel, mesh=mesh,
      compiler_params=pltpu.CompilerParams(use_tc_tiling_on_sc=False),
      out_type=jax.ShapeDtypeStruct((n_sent, seq_pad, PAD_DIM), jnp.float32),
      scratch_types=(
          [pltpu.VMEM((n_sub, SUB), jnp.int32)]
          + [pltpu.VMEM((CHUNK, PAD_DIM), jnp.float32) for _ in range(NBUF)]
          + [pltpu.SemaphoreType.DMA for _ in range(2 * NBUF)]
      ),
  )
  def gather_kernel(table_hbm, idx_hbm, out_hbm, idx_v, *rest):
    rows = rest[:NBUF]
    gsem = rest[NBUF:2 * NBUF]
    ssem = rest[2 * NBUF:]
    wid = lax.axis_index("s") * NC + lax.axis_index("c")
    sent_base = wid * (b_per_w // seq)
    sent_per_chunk = CHUNK // seq

    pltpu.sync_copy(idx_hbm.at[wid], idx_v)

    def start_gather(g, b):
      for k in range(sub_per_chunk):
        pltpu.async_copy(table_hbm.at[idx_v.at[g * sub_per_chunk + k]],
                         rows[b].at[pl.ds(k * SUB, SUB)], gsem[b])

    def wait_gather(g, b):
      for k in range(sub_per_chunk):
        pltpu.make_async_copy(table_hbm.at[idx_v.at[g * sub_per_chunk + k]],
                              rows[b].at[pl.ds(k * SUB, SUB)], gsem[b]).wait()

    def start_store(g, b):
      s0 = sent_base + g * sent_per_chunk
      for s in range(sent_per_chunk):
        pltpu.async_copy(
            rows[b].at[pl.ds(s * seq, seq), pl.ds(0, EMB_DIM)],
            out_hbm.at[s0 + s, pl.ds(0, seq), pl.ds(0, EMB_DIM)], ssem[b])

    def wait_store(g, b):
      s0 = sent_base + g * sent_per_chunk
      for s in range(sent_per_chunk):
        pltpu.make_async_copy(
            rows[b].at[pl.ds(s * seq, seq), pl.ds(0, EMB_DIM)],
            out_hbm.at[s0 + s, pl.ds(0, seq), pl.ds(0, EMB_DIM)],
            ssem[b]).wait()

    for b in range(NBUF):
      start_gather(b, b)

    def body(go, _):
      for b in range(NBUF):
        g = go * NBUF + b
        wait_gather(g, b)
        start_store(g, b)
      for b in range(NBUF):
        g = go * NBUF + b
        wait_store(g, b)
        start_gather(g + NBUF, b)
      return _

    lax.fori_loop(0, n_groups - 1, body, None)

    for b in range(NBUF):
      g = (n_groups - 1) * NBUF + b
      wait_gather(g, b)
      start_store(g, b)
    for b in range(NBUF):
      g = (n_groups - 1) * NBUF + b
      wait_store(g, b)

  return gather_kernel


def kernel(input_, weight):
  info = plsc.get_sparse_core_info()
  NC, NS = info.num_cores, info.num_subcores
  B = input_.size
  wpad = jnp.pad(weight, ((0, 0), (0, PAD_DIM - EMB_DIM)))
  idx = input_.reshape(-1).astype(jnp.int32)
  idx3 = idx.reshape(NC * NS, -1, SUB)
  out = _build(B, input_.shape[1], NC, NS)(wpad, idx3)
  return out[:, :input_.shape[1], :EMB_DIM]


# speedup vs baseline: 1.0097x; 1.0028x over previous
import functools

import jax
import jax.numpy as jnp
from jax import lax
from jax.experimental import pallas as pl
from jax.experimental.pallas import tpu as pltpu
from jax.experimental.pallas import tpu_sc as plsc

EMB_DIM = 64
PAD_DIM = 128
SUB = 80
CHUNK = 160
NBUF = 4


@functools.lru_cache(maxsize=None)
def _build(B, seq, NC, NS):
  NW = NC * NS
  b_per_w = B // NW
  n_sub = b_per_w // SUB
  n_chunks = b_per_w // CHUNK
  sub_per_chunk = CHUNK // SUB
  n_groups = n_chunks // NBUF
  mesh = plsc.VectorSubcoreMesh(core_axis_name="c", subcore_axis_name="s")

  n_sent = B // seq
  seq_pad = (seq + 7) // 8 * 8

  @functools.partial(
      pl.kernel, mesh=mesh,
      compiler_params=pltpu.CompilerParams(use_tc_tiling_on_sc=False),
      out_type=jax.ShapeDtypeStruct((n_sent, seq_pad, PAD_DIM), jnp.float32),
      scratch_types=(
          [pltpu.VMEM((n_sub, SUB), jnp.int32)]
          + [pltpu.VMEM((CHUNK, PAD_DIM), jnp.float32) for _ in range(NBUF)]
          + [pltpu.SemaphoreType.DMA for _ in range(2 * NBUF)]
      ),
  )
  def gather_kernel(table_hbm, idx_hbm, out_hbm, idx_v, *rest):
    rows = rest[:NBUF]
    gsem = rest[NBUF:2 * NBUF]
    ssem = rest[2 * NBUF:]
    wid = lax.axis_index("s") * NC + lax.axis_index("c")
    sent_base = wid * (b_per_w // seq)
    sent_per_chunk = CHUNK // seq

    pltpu.sync_copy(idx_hbm.at[wid], idx_v)

    def start_gather(g, b):
      for k in range(sub_per_chunk):
        pltpu.async_copy(table_hbm.at[idx_v.at[g * sub_per_chunk + k]],
                         rows[b].at[pl.ds(k * SUB, SUB)], gsem[b])

    def wait_gather(g, b):
      for k in range(sub_per_chunk):
        pltpu.make_async_copy(table_hbm.at[idx_v.at[g * sub_per_chunk + k]],
                              rows[b].at[pl.ds(k * SUB, SUB)], gsem[b]).wait()

    def start_store(g, b):
      s0 = sent_base + g * sent_per_chunk
      for s in range(sent_per_chunk):
        pltpu.async_copy(
            rows[b].at[pl.ds(s * seq, seq), pl.ds(0, EMB_DIM)],
            out_hbm.at[s0 + s, pl.ds(0, seq), pl.ds(0, EMB_DIM)], ssem[b])

    def wait_store(g, b):
      s0 = sent_base + g * sent_per_chunk
      for s in range(sent_per_chunk):
        pltpu.make_async_copy(
            rows[b].at[pl.ds(s * seq, seq), pl.ds(0, EMB_DIM)],
            out_hbm.at[s0 + s, pl.ds(0, seq), pl.ds(0, EMB_DIM)],
            ssem[b]).wait()

    for b in range(NBUF):
      start_gather(b, b)

    def body(go, _):
      for b in range(NBUF):
        g = go * NBUF + b
        wait_gather(g, b)
        start_store(g, b)
      for b in range(NBUF):
        g = go * NBUF + b
        wait_store(g, b)
        start_gather(g + NBUF, b)
      return _

    lax.fori_loop(0, n_groups - 1, body, None)

    for b in range(NBUF):
      g = (n_groups - 1) * NBUF + b
      wait_gather(g, b)
      start_store(g, b)
    for b in range(NBUF):
      g = (n_groups - 1) * NBUF + b
      wait_store(g, b)

  return gather_kernel


def kernel(input_, weight):
  info = plsc.get_sparse_core_info()
  NC, NS = info.num_cores, info.num_subcores
  B = input_.size
  wpad = jnp.pad(weight, ((0, 0), (0, PAD_DIM - EMB_DIM)))
  idx = input_.reshape(-1).astype(jnp.int32)
  idx3 = idx.reshape(NC * NS, -1, SUB)
  out = _build(B, input_.shape[1], NC, NS)(wpad, idx3)
  return out[:, :input_.shape[1], :EMB_DIM]


# speedup vs baseline: 1.0282x; 1.0183x over previous
import functools

import jax
import jax.numpy as jnp
from jax import lax
from jax.experimental import pallas as pl
from jax.experimental.pallas import tpu as pltpu
from jax.experimental.pallas import tpu_sc as plsc

EMB_DIM = 64
PAD_DIM = 128
SUB = 80
CHUNK = 80
NBUF = 8


@functools.lru_cache(maxsize=None)
def _build(B, seq, NC, NS):
  NW = NC * NS
  b_per_w = B // NW
  n_sub = b_per_w // SUB
  n_chunks = b_per_w // CHUNK
  sub_per_chunk = CHUNK // SUB
  n_groups = n_chunks // NBUF
  mesh = plsc.VectorSubcoreMesh(core_axis_name="c", subcore_axis_name="s")

  n_sent = B // seq
  seq_pad = (seq + 7) // 8 * 8

  @functools.partial(
      pl.kernel, mesh=mesh,
      compiler_params=pltpu.CompilerParams(use_tc_tiling_on_sc=False),
      out_type=jax.ShapeDtypeStruct((n_sent, seq_pad, PAD_DIM), jnp.float32),
      scratch_types=(
          [pltpu.VMEM((n_sub, SUB), jnp.int32)]
          + [pltpu.VMEM((CHUNK, PAD_DIM), jnp.float32) for _ in range(NBUF)]
          + [pltpu.SemaphoreType.DMA for _ in range(2 * NBUF)]
      ),
  )
  def gather_kernel(table_hbm, idx_hbm, out_hbm, idx_v, *rest):
    rows = rest[:NBUF]
    gsem = rest[NBUF:2 * NBUF]
    ssem = rest[2 * NBUF:]
    wid = lax.axis_index("s") * NC + lax.axis_index("c")
    sent_base = wid * (b_per_w // seq)
    sent_per_chunk = CHUNK // seq

    pltpu.sync_copy(idx_hbm.at[wid], idx_v)

    def start_gather(g, b):
      for k in range(sub_per_chunk):
        pltpu.async_copy(table_hbm.at[idx_v.at[g * sub_per_chunk + k]],
                         rows[b].at[pl.ds(k * SUB, SUB)], gsem[b])

    def wait_gather(g, b):
      for k in range(sub_per_chunk):
        pltpu.make_async_copy(table_hbm.at[idx_v.at[g * sub_per_chunk + k]],
                              rows[b].at[pl.ds(k * SUB, SUB)], gsem[b]).wait()

    def start_store(g, b):
      s0 = sent_base + g * sent_per_chunk
      for s in range(sent_per_chunk):
        pltpu.async_copy(
            rows[b].at[pl.ds(s * seq, seq), pl.ds(0, EMB_DIM)],
            out_hbm.at[s0 + s, pl.ds(0, seq), pl.ds(0, EMB_DIM)], ssem[b])

    def wait_store(g, b):
      s0 = sent_base + g * sent_per_chunk
      for s in range(sent_per_chunk):
        pltpu.make_async_copy(
            rows[b].at[pl.ds(s * seq, seq), pl.ds(0, EMB_DIM)],
            out_hbm.at[s0 + s, pl.ds(0, seq), pl.ds(0, EMB_DIM)],
            ssem[b]).wait()

    for b in range(NBUF):
      start_gather(b, b)

    def body(go, _):
      for b in range(NBUF):
        g = go * NBUF + b
        wait_gather(g, b)
        start_store(g, b)
      for b in range(NBUF):
        g = go * NBUF + b
        wait_store(g, b)
        start_gather(g + NBUF, b)
      return _

    lax.fori_loop(0, n_groups - 1, body, None)

    for b in range(NBUF):
      g = (n_groups - 1) * NBUF + b
      wait_gather(g, b)
      start_store(g, b)
    for b in range(NBUF):
      g = (n_groups - 1) * NBUF + b
      wait_store(g, b)

  return gather_kernel


def kernel(input_, weight):
  info = plsc.get_sparse_core_info()
  NC, NS = info.num_cores, info.num_subcores
  B = input_.size
  wpad = jnp.pad(weight, ((0, 0), (0, PAD_DIM - EMB_DIM)))
  idx = input_.reshape(-1).astype(jnp.int32)
  idx3 = idx.reshape(NC * NS, -1, SUB)
  out = _build(B, input_.shape[1], NC, NS)(wpad, idx3)
  return out[:, :input_.shape[1], :EMB_DIM]
